# Initial kernel scaffold; baseline (speedup 1.0000x reference)
import jax
import jax.numpy as jnp
from jax.experimental import pallas as pl


def kernel(x, W1, b1, W2, b2):
    raise NotImplementedError("write your pallas kernel here")


# speedup vs baseline: 6701.6246x; 6701.6246x over previous
import functools

import jax
import jax.numpy as jnp
from jax.experimental import pallas as pl
from jax.experimental.pallas import tpu as pltpu

_K = 8
_BOUND = 5.0
_SIGMA_MIN = 1e-4
_MIN_DERIV = 1e-3
_CHUNK = 512


def _flow_norm_body(w1_ref, b1_ref, w2_ref, b2_ref, x_ref, o_ref):
    T = x_ref.shape[1]
    C = x_ref.shape[2]
    nck = T // _CHUNK
    f32 = jnp.float32
    Tn = float(T)

    def mom_step(i, carry):
        s1, s2, s3, s4 = carry
        xb = x_ref[0, pl.ds(pl.multiple_of(i * _CHUNK, _CHUNK), _CHUNK), :]
        x2 = xb * xb
        s1 = s1 + jnp.sum(xb, axis=0, keepdims=True)
        s2 = s2 + jnp.sum(x2, axis=0, keepdims=True)
        s3 = s3 + jnp.sum(x2 * xb, axis=0, keepdims=True)
        s4 = s4 + jnp.sum(x2 * x2, axis=0, keepdims=True)
        return s1, s2, s3, s4

    zero = jnp.zeros((1, C), f32)
    s1, s2, s3, s4 = jax.lax.fori_loop(0, nck, mom_step, (zero, zero, zero, zero))

    mu = s1 / Tn
    mu2 = mu * mu
    m2c = s2 - Tn * mu2
    m3c = s3 - 3.0 * mu * s2 + 2.0 * Tn * mu * mu2
    m4c = s4 - 4.0 * mu * s3 + 6.0 * mu2 * s2 - 3.0 * Tn * mu2 * mu2
    sig = jnp.maximum(jnp.sqrt(m2c / (Tn - 1.0)), _SIGMA_MIN)
    inv_sig = 1.0 / sig
    inv_sig2 = inv_sig * inv_sig
    skew = m3c * inv_sig2 * inv_sig / Tn
    ekurt = m4c * inv_sig2 * inv_sig2 / Tn - 3.0

    H = w1_ref.shape[1]
    hid = [
        jnp.maximum(skew * w1_ref[0, h] + ekurt * w1_ref[1, h] + b1_ref[h], 0.0)
        for h in range(H)
    ]
    raw = []
    for o in range(3 * _K + 1):
        acc = zero + b2_ref[o]
        for h in range(H):
            acc = acc + hid[h] * w2_ref[h, o]
        raw.append(acc)

    def softmax_rows(rows):
        m = rows[0]
        for r in rows[1:]:
            m = jnp.maximum(m, r)
        es = [jnp.exp(r - m) for r in rows]
        ssum = es[0]
        for e in es[1:]:
            ssum = ssum + e
        scale = (2.0 * _BOUND) / ssum
        return [e * scale for e in es]

    w = softmax_rows(raw[:_K])
    h = softmax_rows(raw[_K:2 * _K])
    d = []
    for j in range(_K + 1):
        r = raw[2 * _K + j]
        sp = jnp.maximum(r, 0.0) + jnp.log(1.0 + jnp.exp(-jnp.abs(r)))
        d.append(sp + _MIN_DERIV)

    kx = [zero - _BOUND]
    ky = [zero - _BOUND]
    for j in range(_K):
        kx.append(kx[j] + w[j])
        ky.append(ky[j] + h[j])

    invw = [1.0 / jnp.maximum(w[k], 1e-8) for k in range(_K)]
    s_ = [h[k] * invw[k] for k in range(_K)]

    mu_sig = mu * inv_sig
    t = [mu + sig * kx[j] for j in range(_K + 1)]
    t_lo = t[0]
    t_hi = mu + _BOUND * sig

    P = [invw[k] * inv_sig for k in range(_K)]
    Q = [(mu_sig + kx[k]) * invw[k] for k in range(_K)]
    A = [h[k] * s_[k] for k in range(_K)]
    Bc = [h[k] * d[k] for k in range(_K)]
    Cc = s_
    Dc = [d[k + 1] + d[k] - 2.0 * s_[k] for k in range(_K)]

    def deltas(v):
        return [v[0]] + [v[j] - v[j - 1] for j in range(1, _K)]

    Pd = deltas(P)
    Qd = deltas(Q)
    kyd = deltas(ky[:_K])
    Ad = deltas(A)
    Bd = deltas(Bc)
    Cd = deltas(Cc)
    Dd = deltas(Dc)

    def ev_step(i, carry):
        off = pl.multiple_of(i * _CHUNK, _CHUNK)
        xb = x_ref[0, pl.ds(off, _CHUNK), :]
        c = [jnp.where(xb >= t[j], f32(1.0), f32(0.0)) for j in range(1, _K)]

        def sel(dl):
            acc = c[0] * dl[1] + dl[0]
            for j in range(2, _K):
                acc = acc + c[j - 1] * dl[j]
            return acc

        Pv = sel(Pd)
        Qv = sel(Qd)
        kyv = sel(kyd)
        Av = sel(Ad)
        Bv = sel(Bd)
        Cv = sel(Cd)
        Dv = sel(Dd)
        z_ = jnp.clip(Pv * xb - Qv, 0.0, 1.0)
        pq = z_ - z_ * z_
        numer = Av * (z_ * z_) + Bv * pq
        den = jnp.maximum(Cv + Dv * pq, 1e-8)
        y = kyv + numer / den
        ident = (xb - mu) * inv_sig
        y = jnp.where((xb < t_lo) | (xb > t_hi), ident, y)
        o_ref[0, pl.ds(off, _CHUNK), :] = y
        return carry

    jax.lax.fori_loop(0, nck, ev_step, 0)


def kernel(x, W1, b1, W2, b2):
    B, T, C = x.shape
    return pl.pallas_call(
        _flow_norm_body,
        grid=(B,),
        in_specs=[
            pl.BlockSpec(memory_space=pltpu.SMEM),
            pl.BlockSpec(memory_space=pltpu.SMEM),
            pl.BlockSpec(memory_space=pltpu.SMEM),
            pl.BlockSpec(memory_space=pltpu.SMEM),
            pl.BlockSpec((1, T, C), lambda b: (b, 0, 0)),
        ],
        out_specs=pl.BlockSpec((1, T, C), lambda b: (b, 0, 0)),
        out_shape=jax.ShapeDtypeStruct((B, T, C), x.dtype),
        compiler_params=pltpu.CompilerParams(
            dimension_semantics=("arbitrary",),
        ),
    )(W1, b1, W2, b2, x)
